# baseline (speedup 1.0000x reference)
import functools

import jax
import jax.numpy as jnp
from jax import lax
from jax.experimental import pallas as pl
from jax.experimental.pallas import tpu as pltpu
from jax.experimental.pallas import tpu_sc as plsc

_B, _L, _E, _V = 1024, 200, 128, 100000
_NC, _NS = 2, 16
_NW = _NC * _NS
_ROWS_W = _B * _L // _NW
_CHUNK = 128
_NCH = _ROWS_W // _CHUNK
_LANE = 16
_NBUF = 4

_mesh = plsc.VectorSubcoreMesh(core_axis_name="c", subcore_axis_name="s")


@functools.partial(
    pl.kernel,
    out_type=jax.ShapeDtypeStruct((_B * _L, _E), jnp.float32),
    mesh=_mesh,
    scratch_types=(
        [pltpu.VMEM((_NCH, _CHUNK), jnp.int32)]
        + [pltpu.VMEM((_L * _E,), jnp.float32)]
        + [pltpu.VMEM((_CHUNK, _E), jnp.float32)] * _NBUF
        + [pltpu.SemaphoreType.DMA] * (2 * _NBUF + 1)
    ),
)
def _emb_kernel(x_hbm, tok_hbm, pos_hbm, out_hbm, idx_v, pos_v, *scratch):
    bufs = scratch[:_NBUF]
    gsem = scratch[_NBUF:2 * _NBUF]
    ssem = scratch[2 * _NBUF:3 * _NBUF]
    psem = scratch[3 * _NBUF]

    wid = lax.axis_index("s") * _NC + lax.axis_index("c")
    base_row = wid * _ROWS_W

    pltpu.sync_copy(x_hbm.at[wid], idx_v)
    pos_copy = pltpu.async_copy(pos_hbm, pos_v, psem)

    def start_gather(j, b):
        pltpu.async_copy(tok_hbm.at[idx_v.at[j]], bufs[b], gsem[b])

    def wait_gather(j, b):
        pltpu.make_async_copy(tok_hbm.at[idx_v.at[j]], bufs[b], gsem[b]).wait()

    def out_slice(j):
        return out_hbm.at[pl.ds(base_row + j * _CHUNK, _CHUNK)]

    def start_store(j, b):
        pltpu.async_copy(bufs[b], out_slice(j), ssem[b])

    def wait_store(j, b):
        pltpu.make_async_copy(bufs[b], out_slice(j), ssem[b]).wait()

    def add_pos(j, b):
        buf = bufs[b]
        base = j * _CHUNK

        @plsc.parallel_loop(0, _CHUNK, step=1, unroll=4)
        def row_body(i):
            le = lax.rem(base + i, _L) * _E
            for k in range(_E // _LANE):
                buf[i, pl.ds(k * _LANE, _LANE)] += pos_v[
                    pl.ds(le + k * _LANE, _LANE)
                ]

    start_gather(0, 0)
    start_gather(1, 1)
    pos_copy.wait()
    wait_gather(0, 0)
    add_pos(0, 0)
    start_store(0, 0)
    start_gather(2, 2)
    wait_gather(1, 1)
    add_pos(1, 1)
    start_store(1, 1)
    start_gather(3, 3)
    wait_gather(2, 2)
    add_pos(2, 2)
    start_store(2, 2)
    wait_store(0, 0)
    start_gather(4, 0)
    wait_gather(3, 3)
    add_pos(3, 3)
    start_store(3, 3)
    wait_store(1, 1)
    start_gather(5, 1)

    n_outer = (_NCH - 4 - 2) // _NBUF

    def outer(jo, _):
        for bb in range(_NBUF):
            j = 4 + jo * _NBUF + bb
            wait_store(j - 2, (bb + 2) % _NBUF)
            start_gather(j + 2, (bb + 2) % _NBUF)
            wait_gather(j, bb)
            add_pos(j, bb)
            start_store(j, bb)
        return 0

    lax.fori_loop(0, n_outer, outer, 0)

    wait_gather(48, 0)
    add_pos(48, 0)
    start_store(48, 0)
    wait_store(46, 2)
    wait_gather(49, 1)
    add_pos(49, 1)
    start_store(49, 1)
    wait_store(47, 3)
    wait_store(48, 0)
    wait_store(49, 1)


def kernel(x, token_table, pos_table):
    x_flat = x.astype(jnp.int32).reshape(_NW, _NCH, _CHUNK)
    out = _emb_kernel(x_flat, token_table, pos_table.reshape(-1))
    return out.reshape(_B, _L, _E)


# speedup vs baseline: 1.0025x; 1.0025x over previous
import functools

import jax
import jax.numpy as jnp
from jax import lax
from jax.experimental import pallas as pl
from jax.experimental.pallas import tpu as pltpu
from jax.experimental.pallas import tpu_sc as plsc

_B, _L, _E, _V = 1024, 200, 128, 100000
_NC, _NS = 2, 16
_NW = _NC * _NS
_ROWS_W = _B * _L // _NW
_CHUNK = 128
_NCH = _ROWS_W // _CHUNK
_LANE = 16
_NBUF = 4

_mesh = plsc.VectorSubcoreMesh(core_axis_name="c", subcore_axis_name="s")


@functools.partial(
    pl.kernel,
    out_type=jax.ShapeDtypeStruct((_B * _L, _E), jnp.float32),
    mesh=_mesh,
    scratch_types=(
        [pltpu.VMEM((_NCH, _CHUNK), jnp.int32)]
        + [pltpu.VMEM((_L * _E,), jnp.float32)]
        + [pltpu.VMEM((_CHUNK, _E), jnp.float32)] * _NBUF
        + [pltpu.SemaphoreType.DMA] * (2 * _NBUF)
    ),
)
def _emb_kernel(x_hbm, tok_hbm, pos_hbm, out_hbm, idx_v, pos_v, *scratch):
    bufs = scratch[:_NBUF]
    gsem = scratch[_NBUF:2 * _NBUF]
    ssem = scratch[2 * _NBUF:]

    wid = lax.axis_index("s") * _NC + lax.axis_index("c")
    base_row = wid * _ROWS_W

    pltpu.sync_copy(x_hbm.at[wid], idx_v)
    pltpu.sync_copy(pos_hbm, pos_v)

    def start_gather(j, b):
        pltpu.async_copy(tok_hbm.at[idx_v.at[j]], bufs[b], gsem[b])

    def wait_gather(j, b):
        pltpu.make_async_copy(tok_hbm.at[idx_v.at[j]], bufs[b], gsem[b]).wait()

    def out_slice(j):
        return out_hbm.at[pl.ds(base_row + j * _CHUNK, _CHUNK)]

    def start_store(j, b):
        pltpu.async_copy(bufs[b], out_slice(j), ssem[b])

    def wait_store(j, b):
        pltpu.make_async_copy(bufs[b], out_slice(j), ssem[b]).wait()

    def add_pos(j, b):
        buf = bufs[b]
        base = j * _CHUNK

        @plsc.parallel_loop(0, _CHUNK, step=1, unroll=4)
        def row_body(i):
            le = lax.rem(base + i, _L) * _E
            for k in range(_E // _LANE):
                buf[i, pl.ds(k * _LANE, _LANE)] += pos_v[
                    pl.ds(le + k * _LANE, _LANE)
                ]

    start_gather(0, 0)
    start_gather(1, 1)
    wait_gather(0, 0)
    add_pos(0, 0)
    start_store(0, 0)
    start_gather(2, 2)
    wait_gather(1, 1)
    add_pos(1, 1)
    start_store(1, 1)
    start_gather(3, 3)
    wait_gather(2, 2)
    add_pos(2, 2)
    start_store(2, 2)
    wait_store(0, 0)
    start_gather(4, 0)
    wait_gather(3, 3)
    add_pos(3, 3)
    start_store(3, 3)
    wait_store(1, 1)
    start_gather(5, 1)

    n_outer = (_NCH - 4 - 2) // _NBUF

    def outer(jo, _):
        for bb in range(_NBUF):
            j = 4 + jo * _NBUF + bb
            wait_store(j - 2, (bb + 2) % _NBUF)
            start_gather(j + 2, (bb + 2) % _NBUF)
            wait_gather(j, bb)
            add_pos(j, bb)
            start_store(j, bb)
        return 0

    lax.fori_loop(0, n_outer, outer, 0)

    wait_gather(48, 0)
    add_pos(48, 0)
    start_store(48, 0)
    wait_store(46, 2)
    wait_gather(49, 1)
    add_pos(49, 1)
    start_store(49, 1)
    wait_store(47, 3)
    wait_store(48, 0)
    wait_store(49, 1)


def kernel(x, token_table, pos_table):
    x_flat = x.astype(jnp.int32).reshape(_NW, _NCH, _CHUNK)
    out = _emb_kernel(x_flat, token_table, pos_table.reshape(-1))
    return out.reshape(_B, _L, _E)
